# Initial kernel scaffold; baseline (speedup 1.0000x reference)
import functools

import jax
import jax.numpy as jnp
from jax import lax
from jax.experimental import pallas as pl
from jax.experimental.pallas import tpu as pltpu
from jax.experimental.pallas import tpu_sc as plsc

VOCAB = 1000000
POS = 1024
SEG = 64
DIM = 64
B, S = 1024, 200
N = B * S
NC, NS, L = 2, 16, 16
NW = NC * NS
TOK_PER_W = N // NW
CHUNK = 128
NCHUNK = TOK_PER_W // CHUNK


def _sc_kernel(word_hbm, mask_hbm, w_hbm, p2_hbm, sg_hbm, out_hbm,
               widx, midx, wrows, srows, pblk, sem_w, sem_s):
    wid = lax.axis_index("s") * NC + lax.axis_index("c")
    base = wid * TOK_PER_W

    pltpu.sync_copy(p2_hbm, pblk)

    def chunk_body(k, carry):
        off = base + k * CHUNK
        s0 = lax.rem(k * CHUNK, S)

        pltpu.sync_copy(word_hbm.at[pl.ds(off, CHUNK)], widx)
        pltpu.sync_copy(mask_hbm.at[pl.ds(off, CHUNK)], midx)
        cw = pltpu.async_copy(w_hbm.at[widx], wrows, sem_w)
        cs = pltpu.async_copy(sg_hbm.at[midx], srows, sem_s)
        cw.wait()
        cs.wait()

        def tok_body(t, c2):
            for c in range(DIM // L):
                sl = pl.ds(c * L, L)
                wrows[t, sl] = wrows[t, sl] + srows[t, sl] + pblk[s0 + t, sl]
            return c2

        lax.fori_loop(0, CHUNK, tok_body, 0, unroll=2)

        pltpu.sync_copy(wrows, out_hbm.at[pl.ds(off, CHUNK)])
        return carry

    lax.fori_loop(0, NCHUNK, chunk_body, 0)


@jax.jit
def _run(word_flat, mask_flat, W, P2, Sg):
    mesh = plsc.VectorSubcoreMesh(core_axis_name="c", subcore_axis_name="s")
    f = functools.partial(
        pl.kernel,
        mesh=mesh,
        out_type=jax.ShapeDtypeStruct((N, DIM), jnp.float32),
        scratch_types=[
            pltpu.VMEM((CHUNK,), jnp.int32),
            pltpu.VMEM((CHUNK,), jnp.int32),
            pltpu.VMEM((CHUNK, DIM), jnp.float32),
            pltpu.VMEM((CHUNK, DIM), jnp.float32),
            pltpu.VMEM((2 * S, DIM), jnp.float32),
            pltpu.SemaphoreType.DMA,
            pltpu.SemaphoreType.DMA,
        ],
    )(_sc_kernel)
    return f(word_flat, mask_flat, W, P2, Sg)


def kernel(word_ids, mask_ids, W, P, Sg):
    word_flat = word_ids.reshape(-1).astype(jnp.int32)
    mask_flat = mask_ids.reshape(-1).astype(jnp.int32)
    P200 = P[:S].astype(jnp.float32)
    P2 = jnp.concatenate([P200, P200], axis=0)
    out = _run(word_flat, mask_flat, W, P2, Sg)
    return out.reshape(B, S, DIM)


# speedup vs baseline: 1.6850x; 1.6850x over previous
import functools

import jax
import jax.numpy as jnp
from jax import lax
from jax.experimental import pallas as pl
from jax.experimental.pallas import tpu as pltpu
from jax.experimental.pallas import tpu_sc as plsc

VOCAB = 1000000
POS = 1024
SEG = 64
DIM = 64
B, S = 1024, 200
N = B * S
NC, NS, L = 2, 16, 16
NW = NC * NS
TOK_PER_W = N // NW
CHUNK = 128
NCHUNK = TOK_PER_W // CHUNK


def _sc_kernel(word_hbm, mask_hbm, w_hbm, p2_hbm, sg_hbm, out_hbm,
               widx, midx, wrows, srows, pblk, sem_w, sem_s):
    wid = lax.axis_index("s") * NC + lax.axis_index("c")
    base = wid * TOK_PER_W

    pltpu.sync_copy(p2_hbm, pblk)

    def chunk_body(k, carry):
        off = base + k * CHUNK
        s0 = lax.rem(k * CHUNK, S)

        pltpu.sync_copy(word_hbm.at[pl.ds(off, CHUNK)], widx)
        pltpu.sync_copy(mask_hbm.at[pl.ds(off, CHUNK)], midx)
        cw = pltpu.async_copy(w_hbm.at[widx], wrows, sem_w)
        cs = pltpu.async_copy(sg_hbm.at[midx], srows, sem_s)
        cw.wait()
        cs.wait()

        def tok_body(t, c2):
            for c in range(DIM // L):
                sl = pl.ds(c * L, L)
                wrows[t, sl] = wrows[t, sl] + srows[t, sl] + pblk[s0 + t, sl]
            return c2

        lax.fori_loop(0, CHUNK, tok_body, 0, unroll=2)

        pltpu.sync_copy(wrows, out_hbm.at[pl.ds(off, CHUNK)])
        return carry

    lax.fori_loop(0, NCHUNK, chunk_body, 0)


@jax.jit
def _run(word_flat, mask_flat, W, P2, Sg):
    mesh = plsc.VectorSubcoreMesh(core_axis_name="c", subcore_axis_name="s")
    f = functools.partial(
        pl.kernel,
        mesh=mesh,
        compiler_params=pltpu.CompilerParams(use_tc_tiling_on_sc=False),
        out_type=jax.ShapeDtypeStruct((N, DIM), jnp.float32),
        scratch_types=[
            pltpu.VMEM((CHUNK,), jnp.int32),
            pltpu.VMEM((CHUNK,), jnp.int32),
            pltpu.VMEM((CHUNK, DIM), jnp.float32),
            pltpu.VMEM((CHUNK, DIM), jnp.float32),
            pltpu.VMEM((2 * S, DIM), jnp.float32),
            pltpu.SemaphoreType.DMA,
            pltpu.SemaphoreType.DMA,
        ],
    )(_sc_kernel)
    return f(word_flat, mask_flat, W, P2, Sg)


def kernel(word_ids, mask_ids, W, P, Sg):
    word_flat = word_ids.reshape(-1).astype(jnp.int32)
    mask_flat = mask_ids.reshape(-1).astype(jnp.int32)
    P200 = P[:S].astype(jnp.float32)
    P2 = jnp.concatenate([P200, P200], axis=0)
    out = _run(word_flat, mask_flat, W, P2, Sg)
    return out.reshape(B, S, DIM)


# speedup vs baseline: 2.4491x; 1.4535x over previous
import functools

import jax
import jax.numpy as jnp
from jax import lax
from jax.experimental import pallas as pl
from jax.experimental.pallas import tpu as pltpu
from jax.experimental.pallas import tpu_sc as plsc

VOCAB = 1000000
SEG = 64
DIM = 64
B, S = 1024, 200
N = B * S
NC, NS, L = 2, 16, 16
NW = NC * NS
TOK_PER_W = N // NW
CHUNK = 128
NCHUNK = TOK_PER_W // CHUNK


def _sc_kernel(word_hbm, mask_hbm, w_hbm, p2_hbm, sg_hbm, out_hbm,
               widx, midx, wrows, pblk, sgblk, sem_g):
    wid = lax.axis_index("s") * NC + lax.axis_index("c")
    base = wid * TOK_PER_W

    pltpu.sync_copy(p2_hbm, pblk)
    pltpu.sync_copy(sg_hbm, sgblk)

    def chunk_body(k, carry):
        off = base + k * CHUNK
        s0 = lax.rem(k * CHUNK, S)
        pltpu.sync_copy(word_hbm.at[pl.ds(off, CHUNK)], widx)
        pltpu.sync_copy(mask_hbm.at[pl.ds(off, CHUNK)], midx)

        def fire(t16, c2):
            v = widx[pl.ds(t16 * L, L)]
            for j in range(L):
                pltpu.async_copy(w_hbm.at[v[j]], wrows.at[t16 * L + j], sem_g)
            return c2
        lax.fori_loop(0, CHUNK // L, fire, 0)
        pltpu.make_async_copy(w_hbm.at[pl.ds(0, CHUNK)], wrows, sem_g).wait()

        def tok_body(t16, c2):
            mv = midx[pl.ds(t16 * L, L)]
            for j in range(L):
                t = t16 * L + j
                m = mv[j]
                for c in range(DIM // L):
                    sl = pl.ds(c * L, L)
                    wrows[t, sl] = (wrows[t, sl] + sgblk[m, sl]
                                    + pblk[s0 + t, sl])
            return c2
        lax.fori_loop(0, CHUNK // L, tok_body, 0)

        pltpu.sync_copy(wrows, out_hbm.at[pl.ds(off, CHUNK)])
        return carry

    lax.fori_loop(0, NCHUNK, chunk_body, 0)


@jax.jit
def _run(word_flat, mask_flat, W, P2, Sg2):
    mesh = plsc.VectorSubcoreMesh(core_axis_name="c", subcore_axis_name="s")
    f = functools.partial(
        pl.kernel,
        mesh=mesh,
        compiler_params=pltpu.CompilerParams(use_tc_tiling_on_sc=True),
        out_type=jax.ShapeDtypeStruct((N, DIM), jnp.float32),
        scratch_types=[
            pltpu.VMEM((CHUNK,), jnp.int32),
            pltpu.VMEM((CHUNK,), jnp.int32),
            pltpu.VMEM((CHUNK, DIM), jnp.float32),
            pltpu.VMEM((2 * S, DIM), jnp.float32),
            pltpu.VMEM((SEG, DIM), jnp.float32),
            pltpu.SemaphoreType.DMA,
        ],
    )(_sc_kernel)
    return f(word_flat, mask_flat, W, P2, Sg2)


def kernel(word_ids, mask_ids, W, P, Sg):
    word_flat = word_ids.reshape(-1).astype(jnp.int32)
    mask_flat = mask_ids.reshape(-1).astype(jnp.int32)
    P200 = P[:S].astype(jnp.float32)
    P2 = jnp.concatenate([P200, P200], axis=0)
    out = _run(word_flat, mask_flat, W, P2, Sg)
    return out.reshape(B, S, DIM)


# speedup vs baseline: 2.7235x; 1.1120x over previous
import functools

import jax
import jax.numpy as jnp
from jax import lax
from jax.experimental import pallas as pl
from jax.experimental.pallas import tpu as pltpu
from jax.experimental.pallas import tpu_sc as plsc

VOCAB = 1000000
SEG = 64
DIM = 64
B, S = 1024, 200
N = B * S
NC, NS, L = 2, 16, 16
NW = NC * NS
TOK_PER_W = N // NW
CHUNK = 128
NCHUNK = TOK_PER_W // CHUNK
WPAD = 2 * DIM


def _sc_kernel(word_hbm, mask_hbm, wp_hbm, p2_hbm, sg_hbm, out_hbm,
               widx0, widx1, midx0, midx1,
               wrows0, wrows1, obuf0, obuf1, pblk, sgblk,
               sem_g0, sem_g1, sem_o0, sem_o1):
    wid = lax.axis_index("s") * NC + lax.axis_index("c")
    base = wid * TOK_PER_W

    pltpu.sync_copy(p2_hbm, pblk)
    pltpu.sync_copy(sg_hbm, sgblk)

    widx = (widx0, widx1)
    midx = (midx0, midx1)
    wrows = (wrows0, wrows1)
    obuf = (obuf0, obuf1)
    sem_g = (sem_g0, sem_g1)
    sem_o = (sem_o0, sem_o1)

    def prefetch(k, b):
        off = base + k * CHUNK
        pltpu.sync_copy(word_hbm.at[pl.ds(off, CHUNK)], widx[b])
        pltpu.sync_copy(mask_hbm.at[pl.ds(off, CHUNK)], midx[b])

        def fire(t16, c2):
            v = widx[b][pl.ds(t16 * L, L)]
            for j in range(L):
                pltpu.async_copy(wp_hbm.at[v[j]], wrows[b].at[t16 * L + j],
                                 sem_g[b])
            return c2
        lax.fori_loop(0, CHUNK // L, fire, 0)

    def drain_gather(b):
        pltpu.make_async_copy(wp_hbm.at[pl.ds(0, CHUNK)], wrows[b],
                              sem_g[b]).wait()

    def compute(k, b):
        s0 = lax.rem(k * CHUNK, S)

        def tok_body(t16, c2):
            mv = midx[b][pl.ds(t16 * L, L)]
            for j in range(L):
                t = t16 * L + j
                m = mv[j]
                for c in range(DIM // L):
                    sl = pl.ds(c * L, L)
                    obuf[b][t, sl] = (wrows[b][t, sl]
                                      + sgblk[m, sl] + pblk[s0 + t, sl])
            return c2
        lax.fori_loop(0, CHUNK // L, tok_body, 0)

    def store(k, b):
        pltpu.async_copy(obuf[b], out_hbm.at[pl.ds(base + k * CHUNK, CHUNK)],
                         sem_o[b])

    def wait_store(b):
        pltpu.make_async_copy(out_hbm.at[pl.ds(0, CHUNK)], obuf[b],
                              sem_o[b]).wait()

    prefetch(0, 0)

    def chunk_body(k, carry):
        b = lax.rem(k, 2)
        for bb in range(2):
            @pl.when((b == bb) & (k + 1 < NCHUNK))
            def _():
                prefetch(k + 1, 1 - bb)

        for bb in range(2):
            @pl.when(b == bb)
            def _():
                drain_gather(bb)

                @pl.when(k >= 2)
                def _():
                    wait_store(bb)
                compute(k, bb)
                store(k, bb)
        return carry

    lax.fori_loop(0, NCHUNK, chunk_body, 0)
    wait_store(0)
    wait_store(1)


@jax.jit
def _run(word_flat, mask_flat, Wp, P2, Sg2):
    mesh = plsc.VectorSubcoreMesh(core_axis_name="c", subcore_axis_name="s")
    f = functools.partial(
        pl.kernel,
        mesh=mesh,
        compiler_params=pltpu.CompilerParams(use_tc_tiling_on_sc=True),
        out_type=jax.ShapeDtypeStruct((N, DIM), jnp.float32),
        scratch_types=[
            pltpu.VMEM((CHUNK,), jnp.int32),
            pltpu.VMEM((CHUNK,), jnp.int32),
            pltpu.VMEM((CHUNK,), jnp.int32),
            pltpu.VMEM((CHUNK,), jnp.int32),
            pltpu.VMEM((CHUNK, DIM), jnp.float32),
            pltpu.VMEM((CHUNK, DIM), jnp.float32),
            pltpu.VMEM((CHUNK, DIM), jnp.float32),
            pltpu.VMEM((CHUNK, DIM), jnp.float32),
            pltpu.VMEM((2 * S, DIM), jnp.float32),
            pltpu.VMEM((SEG, DIM), jnp.float32),
            pltpu.SemaphoreType.DMA,
            pltpu.SemaphoreType.DMA,
            pltpu.SemaphoreType.DMA,
            pltpu.SemaphoreType.DMA,
        ],
    )(_sc_kernel)
    return f(word_flat, mask_flat, Wp, P2, Sg2)


def kernel(word_ids, mask_ids, W, P, Sg):
    word_flat = word_ids.reshape(-1).astype(jnp.int32)
    mask_flat = mask_ids.reshape(-1).astype(jnp.int32)
    Wp = W
    P200 = P[:S].astype(jnp.float32)
    P2 = jnp.concatenate([P200, P200], axis=0)
    out = _run(word_flat, mask_flat, Wp, P2, Sg)
    return out.reshape(B, S, DIM)


# speedup vs baseline: 2.8098x; 1.0317x over previous
import functools

import jax
import jax.numpy as jnp
from jax import lax
from jax.experimental import pallas as pl
from jax.experimental.pallas import tpu as pltpu
from jax.experimental.pallas import tpu_sc as plsc

VOCAB = 1000000
SEG = 64
DIM = 64
B, S = 1024, 200
N = B * S
NC, NS, L = 2, 16, 16
NW = NC * NS
TOK_PER_W = N // NW
CHUNK = 128
NCHUNK = TOK_PER_W // CHUNK
CW = 2 * DIM


def _sc_kernel(word_hbm, cidx_hbm, w_hbm, c_hbm, out_hbm,
               widx0, widx1, cidx0, cidx1,
               wrows0, wrows1, crows0, crows1, obuf0, obuf1,
               sem_g0, sem_g1, sem_c0, sem_c1, sem_o0, sem_o1):
    wid = lax.axis_index("s") * NC + lax.axis_index("c")
    base = wid * TOK_PER_W

    widx = (widx0, widx1)
    cidx = (cidx0, cidx1)
    wrows = (wrows0, wrows1)
    crows = (crows0, crows1)
    obuf = (obuf0, obuf1)
    sem_g = (sem_g0, sem_g1)
    sem_c = (sem_c0, sem_c1)
    sem_o = (sem_o0, sem_o1)

    def prefetch(k, b):
        off = base + k * CHUNK
        pltpu.sync_copy(word_hbm.at[pl.ds(off, CHUNK)], widx[b])
        pltpu.sync_copy(cidx_hbm.at[pl.ds(off, CHUNK)], cidx[b])
        pltpu.async_copy(c_hbm.at[cidx[b]], crows[b], sem_c[b])

        def fire(t16, c2):
            v = widx[b][pl.ds(t16 * L, L)]
            for j in range(L):
                pltpu.async_copy(w_hbm.at[v[j]], wrows[b].at[t16 * L + j],
                                 sem_g[b])
            return c2
        lax.fori_loop(0, CHUNK // L, fire, 0)

    def drain_gather(b):
        pltpu.make_async_copy(w_hbm.at[pl.ds(0, CHUNK)], wrows[b],
                              sem_g[b]).wait()
        pltpu.make_async_copy(c_hbm.at[pl.ds(0, CHUNK)], crows[b],
                              sem_c[b]).wait()

    def compute(b):
        def tok_body(t, c2):
            for c in range(DIM // L):
                sl = pl.ds(c * L, L)
                obuf[b][t, sl] = wrows[b][t, sl] + crows[b][t, sl]
            return c2
        lax.fori_loop(0, CHUNK, tok_body, 0, unroll=4)

    def store(k, b):
        pltpu.async_copy(obuf[b], out_hbm.at[pl.ds(base + k * CHUNK, CHUNK)],
                         sem_o[b])

    def wait_store(b):
        pltpu.make_async_copy(out_hbm.at[pl.ds(0, CHUNK)], obuf[b],
                              sem_o[b]).wait()

    prefetch(0, 0)

    def chunk_body(k, carry):
        b = lax.rem(k, 2)
        for bb in range(2):
            @pl.when((b == bb) & (k + 1 < NCHUNK))
            def _():
                prefetch(k + 1, 1 - bb)

        for bb in range(2):
            @pl.when(b == bb)
            def _():
                drain_gather(bb)

                @pl.when(k >= 2)
                def _():
                    wait_store(bb)
                compute(bb)
                store(k, bb)
        return carry

    lax.fori_loop(0, NCHUNK, chunk_body, 0)
    wait_store(0)
    wait_store(1)


@jax.jit
def _run(word_flat, cidx_flat, W, Cpad):
    mesh = plsc.VectorSubcoreMesh(core_axis_name="c", subcore_axis_name="s")
    f = functools.partial(
        pl.kernel,
        mesh=mesh,
        compiler_params=pltpu.CompilerParams(use_tc_tiling_on_sc=True),
        out_type=jax.ShapeDtypeStruct((N, DIM), jnp.float32),
        scratch_types=[
            pltpu.VMEM((CHUNK,), jnp.int32),
            pltpu.VMEM((CHUNK,), jnp.int32),
            pltpu.VMEM((CHUNK,), jnp.int32),
            pltpu.VMEM((CHUNK,), jnp.int32),
            pltpu.VMEM((CHUNK, DIM), jnp.float32),
            pltpu.VMEM((CHUNK, DIM), jnp.float32),
            pltpu.VMEM((CHUNK, CW), jnp.float32),
            pltpu.VMEM((CHUNK, CW), jnp.float32),
            pltpu.VMEM((CHUNK, DIM), jnp.float32),
            pltpu.VMEM((CHUNK, DIM), jnp.float32),
            pltpu.SemaphoreType.DMA,
            pltpu.SemaphoreType.DMA,
            pltpu.SemaphoreType.DMA,
            pltpu.SemaphoreType.DMA,
            pltpu.SemaphoreType.DMA,
            pltpu.SemaphoreType.DMA,
        ],
    )(_sc_kernel)
    return f(word_flat, cidx_flat, W, Cpad)


def kernel(word_ids, mask_ids, W, P, Sg):
    word_flat = word_ids.reshape(-1).astype(jnp.int32)
    mask_flat = mask_ids.reshape(-1).astype(jnp.int32)
    pos_flat = jnp.broadcast_to(jnp.arange(S, dtype=jnp.int32),
                                (B, S)).reshape(-1)
    cidx_flat = mask_flat * S + pos_flat
    C = (Sg[:, None, :] + P[None, :S, :]).reshape(SEG * S, DIM)
    Cpad = jnp.pad(C, ((0, 0), (0, CW - DIM)))
    out = _run(word_flat, cidx_flat, W, Cpad)
    return out.reshape(B, S, DIM)


# speedup vs baseline: 3.0739x; 1.0940x over previous
import functools

import jax
import jax.numpy as jnp
from jax import lax
from jax.experimental import pallas as pl
from jax.experimental.pallas import tpu as pltpu
from jax.experimental.pallas import tpu_sc as plsc

VOCAB = 1000000
SEG = 64
DIM = 64
B, S = 1024, 200
N = B * S
NC, NS, L = 2, 16, 16
NW = NC * NS
TOK_PER_W = N // NW
CHUNK = 128
NCHUNK = TOK_PER_W // CHUNK
CW = 2 * DIM


def _sc_kernel(word_hbm, cidx_hbm, w_hbm, c_hbm, out_hbm,
               widx_all, cidx_all,
               wrows0, wrows1, crows0, crows1, obuf0, obuf1,
               sem_g0, sem_g1, sem_c0, sem_c1, sem_o0, sem_o1):
    wid = lax.axis_index("s") * NC + lax.axis_index("c")
    base = wid * TOK_PER_W

    wrows = (wrows0, wrows1)
    crows = (crows0, crows1)
    obuf = (obuf0, obuf1)
    sem_g = (sem_g0, sem_g1)
    sem_c = (sem_c0, sem_c1)
    sem_o = (sem_o0, sem_o1)

    pltpu.sync_copy(word_hbm.at[pl.ds(base, TOK_PER_W)], widx_all)
    pltpu.sync_copy(cidx_hbm.at[pl.ds(base, TOK_PER_W)], cidx_all)

    def prefetch(k, b):
        pltpu.async_copy(c_hbm.at[cidx_all.at[pl.ds(k * CHUNK, CHUNK)]],
                         crows[b], sem_c[b])

        def fire(t16, c2):
            v = widx_all[pl.ds(k * CHUNK + t16 * L, L)]
            for j in range(L):
                pltpu.async_copy(w_hbm.at[v[j]], wrows[b].at[t16 * L + j],
                                 sem_g[b])
            return c2
        lax.fori_loop(0, CHUNK // L, fire, 0)

    def drain_gather(b):
        pltpu.make_async_copy(w_hbm.at[pl.ds(0, CHUNK)], wrows[b],
                              sem_g[b]).wait()
        pltpu.make_async_copy(c_hbm.at[pl.ds(0, CHUNK)], crows[b],
                              sem_c[b]).wait()

    def compute(b):
        def tok_body(t, c2):
            for c in range(DIM // L):
                sl = pl.ds(c * L, L)
                obuf[b][t, sl] = wrows[b][t, sl] + crows[b][t, sl]
            return c2
        lax.fori_loop(0, CHUNK, tok_body, 0, unroll=4)

    def store(k, b):
        pltpu.async_copy(obuf[b], out_hbm.at[pl.ds(base + k * CHUNK, CHUNK)],
                         sem_o[b])

    def wait_store(b):
        pltpu.make_async_copy(out_hbm.at[pl.ds(0, CHUNK)], obuf[b],
                              sem_o[b]).wait()

    prefetch(0, 0)

    def chunk_body(k, carry):
        b = lax.rem(k, 2)
        for bb in range(2):
            @pl.when((b == bb) & (k + 1 < NCHUNK))
            def _():
                prefetch(k + 1, 1 - bb)

        for bb in range(2):
            @pl.when(b == bb)
            def _():
                drain_gather(bb)

                @pl.when(k >= 2)
                def _():
                    wait_store(bb)
                compute(bb)
                store(k, bb)
        return carry

    lax.fori_loop(0, NCHUNK, chunk_body, 0)
    wait_store(0)
    wait_store(1)


@jax.jit
def _run(word_flat, cidx_flat, W, Cpad):
    mesh = plsc.VectorSubcoreMesh(core_axis_name="c", subcore_axis_name="s")
    f = functools.partial(
        pl.kernel,
        mesh=mesh,
        compiler_params=pltpu.CompilerParams(use_tc_tiling_on_sc=True),
        out_type=jax.ShapeDtypeStruct((N, DIM), jnp.float32),
        scratch_types=[
            pltpu.VMEM((TOK_PER_W,), jnp.int32),
            pltpu.VMEM((TOK_PER_W,), jnp.int32),
            pltpu.VMEM((CHUNK, DIM), jnp.float32),
            pltpu.VMEM((CHUNK, DIM), jnp.float32),
            pltpu.VMEM((CHUNK, CW), jnp.float32),
            pltpu.VMEM((CHUNK, CW), jnp.float32),
            pltpu.VMEM((CHUNK, DIM), jnp.float32),
            pltpu.VMEM((CHUNK, DIM), jnp.float32),
            pltpu.SemaphoreType.DMA,
            pltpu.SemaphoreType.DMA,
            pltpu.SemaphoreType.DMA,
            pltpu.SemaphoreType.DMA,
            pltpu.SemaphoreType.DMA,
            pltpu.SemaphoreType.DMA,
        ],
    )(_sc_kernel)
    return f(word_flat, cidx_flat, W, Cpad)


def kernel(word_ids, mask_ids, W, P, Sg):
    word_flat = word_ids.reshape(-1).astype(jnp.int32)
    mask_flat = mask_ids.reshape(-1).astype(jnp.int32)
    pos_flat = jnp.broadcast_to(jnp.arange(S, dtype=jnp.int32),
                                (B, S)).reshape(-1)
    cidx_flat = mask_flat * S + pos_flat
    C = (Sg[:, None, :] + P[None, :S, :]).reshape(SEG * S, DIM)
    Cpad = jnp.pad(C, ((0, 0), (0, CW - DIM)))
    out = _run(word_flat, cidx_flat, W, Cpad)
    return out.reshape(B, S, DIM)


# speedup vs baseline: 3.3560x; 1.0917x over previous
import functools

import jax
import jax.numpy as jnp
from jax import lax
from jax.experimental import pallas as pl
from jax.experimental.pallas import tpu as pltpu
from jax.experimental.pallas import tpu_sc as plsc

VOCAB = 1000000
SEG = 64
DIM = 64
B, S = 1024, 200
N = B * S
NC, NS, L = 2, 16, 16
NW = NC * NS
TOK_PER_W = N // NW
CHUNK = 128
NCHUNK = TOK_PER_W // CHUNK
CW = 2 * DIM


def _sc_kernel(word_hbm, cidx_hbm, w_hbm, c_hbm, out_hbm,
               widx_all, cidx_all,
               crows0, crows1, crows2, obuf0, obuf1, obuf2,
               sem_c0, sem_c1, sem_c2, sem_g0, sem_g1, sem_g2,
               sem_o0, sem_o1, sem_o2):
    wid = lax.axis_index("s") * NC + lax.axis_index("c")
    base = wid * TOK_PER_W

    crows = (crows0, crows1, crows2)
    obuf = (obuf0, obuf1, obuf2)
    sem_c = (sem_c0, sem_c1, sem_c2)
    sem_g = (sem_g0, sem_g1, sem_g2)
    sem_o = (sem_o0, sem_o1, sem_o2)

    pltpu.sync_copy(word_hbm.at[pl.ds(base, TOK_PER_W)], widx_all)
    pltpu.sync_copy(cidx_hbm.at[pl.ds(base, TOK_PER_W)], cidx_all)

    def prefetch(k, b):
        pltpu.async_copy(c_hbm.at[cidx_all.at[pl.ds(k * CHUNK, CHUNK)]],
                         crows[b], sem_c[b])

        def fire(t16, c2):
            v = widx_all[pl.ds(k * CHUNK + t16 * L, L)]
            for j in range(L):
                pltpu.async_copy(w_hbm.at[v[j]], obuf[b].at[t16 * L + j],
                                 sem_g[b])
            return c2
        lax.fori_loop(0, CHUNK // L, fire, 0)

    def drain_gather(b):
        pltpu.make_async_copy(w_hbm.at[pl.ds(0, CHUNK)], obuf[b],
                              sem_g[b]).wait()
        pltpu.make_async_copy(c_hbm.at[pl.ds(0, CHUNK)], crows[b],
                              sem_c[b]).wait()

    def compute(b):
        def tok_body(t, c2):
            for c in range(DIM // L):
                sl = pl.ds(c * L, L)
                plsc.addupdate(obuf[b].at[t, sl], crows[b][t, sl])
            return c2
        lax.fori_loop(0, CHUNK, tok_body, 0, unroll=4)

    def store(k, b):
        pltpu.async_copy(obuf[b], out_hbm.at[pl.ds(base + k * CHUNK, CHUNK)],
                         sem_o[b])

    def wait_store(b):
        pltpu.make_async_copy(out_hbm.at[pl.ds(0, CHUNK)], obuf[b],
                              sem_o[b]).wait()

    prefetch(0, 0)

    def chunk_body(k, carry):
        b = lax.rem(k, 3)
        for p in range(3):
            pn = (p + 1) % 3

            @pl.when((b == p) & (k + 1 < NCHUNK))
            def _():
                @pl.when(k >= 2)
                def _():
                    wait_store(pn)
                prefetch(k + 1, pn)

        for p in range(3):
            @pl.when(b == p)
            def _():
                drain_gather(p)
                compute(p)
                store(k, p)
        return carry

    lax.fori_loop(0, NCHUNK, chunk_body, 0)
    wait_store(0)
    wait_store(1)
    wait_store(2)


@jax.jit
def _run(word_flat, cidx_flat, W, Cpad):
    mesh = plsc.VectorSubcoreMesh(core_axis_name="c", subcore_axis_name="s")
    f = functools.partial(
        pl.kernel,
        mesh=mesh,
        compiler_params=pltpu.CompilerParams(use_tc_tiling_on_sc=True),
        out_type=jax.ShapeDtypeStruct((N, DIM), jnp.float32),
        scratch_types=[
            pltpu.VMEM((TOK_PER_W,), jnp.int32),
            pltpu.VMEM((TOK_PER_W,), jnp.int32),
            pltpu.VMEM((CHUNK, CW), jnp.float32),
            pltpu.VMEM((CHUNK, CW), jnp.float32),
            pltpu.VMEM((CHUNK, CW), jnp.float32),
            pltpu.VMEM((CHUNK, DIM), jnp.float32),
            pltpu.VMEM((CHUNK, DIM), jnp.float32),
            pltpu.VMEM((CHUNK, DIM), jnp.float32),
            pltpu.SemaphoreType.DMA,
            pltpu.SemaphoreType.DMA,
            pltpu.SemaphoreType.DMA,
            pltpu.SemaphoreType.DMA,
            pltpu.SemaphoreType.DMA,
            pltpu.SemaphoreType.DMA,
            pltpu.SemaphoreType.DMA,
            pltpu.SemaphoreType.DMA,
            pltpu.SemaphoreType.DMA,
        ],
    )(_sc_kernel)
    return f(word_flat, cidx_flat, W, Cpad)


def kernel(word_ids, mask_ids, W, P, Sg):
    word_flat = word_ids.reshape(-1).astype(jnp.int32)
    mask_flat = mask_ids.reshape(-1).astype(jnp.int32)
    pos_flat = jnp.broadcast_to(jnp.arange(S, dtype=jnp.int32),
                                (B, S)).reshape(-1)
    cidx_flat = mask_flat * S + pos_flat
    C = (Sg[:, None, :] + P[None, :S, :]).reshape(SEG * S, DIM)
    Cpad = jnp.pad(C, ((0, 0), (0, CW - DIM)))
    out = _run(word_flat, cidx_flat, W, Cpad)
    return out.reshape(B, S, DIM)
